# Initial kernel scaffold; baseline (speedup 1.0000x reference)
import jax
import jax.numpy as jnp
from jax.experimental import pallas as pl


def kernel(x, table, W, b):
    raise NotImplementedError("write your pallas kernel here")


# speedup vs baseline: 4.8270x; 4.8270x over previous
import functools

import jax
import jax.numpy as jnp
from jax import lax
from jax.experimental import pallas as pl
from jax.experimental.pallas import tpu as pltpu
from jax.experimental.pallas import tpu_sc as plsc

VOCAB = 148
P = 50
D = 128
B = 4096
VPAD = 160
NC, NS = 2, 16
NW = NC * NS
BPW = B // NW
LANES = 16


def _proj_body(table_ref, w_ref, b_ref, out_ref):
    p = pl.program_id(0)
    acc = jnp.dot(table_ref[...], w_ref[0],
                  preferred_element_type=jnp.float32)
    scale = jnp.where(p == 0, 1.0, 0.0).astype(jnp.float32)
    out_ref[0] = acc + scale * b_ref[0]


def _build_m(table_pad, w3, bias_row):
    return pl.pallas_call(
        _proj_body,
        grid=(P,),
        in_specs=[
            pl.BlockSpec((VPAD, D), lambda p: (0, 0)),
            pl.BlockSpec((1, D, D), lambda p: (p, 0, 0)),
            pl.BlockSpec((1, D), lambda p: (0, 0)),
        ],
        out_specs=pl.BlockSpec((1, VPAD, D), lambda p: (p, 0, 0)),
        out_shape=jax.ShapeDtypeStruct((P, VPAD, D), jnp.float32),
    )(table_pad, w3, bias_row)


_mesh = plsc.VectorSubcoreMesh(core_axis_name="c", subcore_axis_name="s")


@functools.partial(
    pl.kernel,
    mesh=_mesh,
    out_type=jax.ShapeDtypeStruct((B, D), jnp.float32),
    scratch_types=[
        pltpu.VMEM((P, BPW), jnp.int32),
        pltpu.VMEM((BPW, D), jnp.float32),
        pltpu.VMEM((BPW, D), jnp.float32),
        pltpu.SemaphoreType.DMA,
    ],
)
def _sc_gather_sum(m_hbm, idx_hbm, out_hbm, idx_v, rows_v, acc_v, sem):
    c = lax.axis_index("c")
    s = lax.axis_index("s")
    wid = s * NC + c

    pltpu.sync_copy(idx_hbm.at[wid], idx_v)

    pltpu.async_copy(m_hbm.at[idx_v.at[0]], acc_v, sem).wait()

    def pos_body(j, carry):
        pltpu.async_copy(m_hbm.at[idx_v.at[j]], rows_v, sem).wait()

        def row_body(i, c2):
            for k in range(D // LANES):
                sl = pl.ds(k * LANES, LANES)
                plsc.addupdate(acc_v.at[i, sl], rows_v[i, sl])
            return c2

        lax.fori_loop(0, BPW, row_body, 0)
        return carry

    lax.fori_loop(1, P, pos_body, 0)

    pltpu.sync_copy(acc_v, out_hbm.at[pl.ds(wid * BPW, BPW)])


def kernel(x, table, W, b):
    table_pad = jnp.zeros((VPAD, D), jnp.float32).at[:VOCAB].set(table)
    w3 = W.reshape(P, D, D)
    m = _build_m(table_pad, w3, b.reshape(1, D)).reshape(P * VPAD, D)

    xw = x.astype(jnp.int32).reshape(NW, BPW, P).transpose(0, 2, 1)
    fidx = xw + (jnp.arange(P, dtype=jnp.int32) * VPAD)[None, :, None]

    return _sc_gather_sum(m, fidx)
